# baseline (speedup 1.0000x reference)
import jax
import jax.numpy as jnp
from jax import lax
from jax.experimental import pallas as pl
from jax.experimental.pallas import tpu as pltpu

_TJ = 128
_TI = 256


def _gat_body(x_ref, adj_ref, wlr_ref, att_ref, bias_ref, out_ref):
    n = x_ref.shape[1]
    dh = att_ref.shape[2]
    x = x_ref[0]
    wlr = wlr_ref[0]
    att = att_ref[0]

    xlr = jnp.dot(x, wlr, preferred_element_type=jnp.float32)
    xl = xlr[:, :dh]
    xr = xlr[:, dh:]

    xla_s = xl * (0.4 * att)
    xra_s = xr * (0.4 * att)
    al2 = 1.5 * jnp.sum(xla_s, axis=1, keepdims=True)
    a_l1 = jnp.sum(jnp.abs(xla_s), axis=1, keepdims=True)
    kmax = jnp.max(al2 + a_l1)
    xrab_t = lax.transpose(xra_s, (1, 0))
    ar2 = 1.5 * jnp.sum(xrab_t, axis=0, keepdims=True)
    c_l1 = jnp.sum(jnp.abs(xrab_t), axis=0, keepdims=True)
    mrow = -(kmax + c_l1)

    xlab = xla_s.astype(jnp.bfloat16)
    xrab = xrab_t.astype(jnp.bfloat16)
    xl_aug = jnp.concatenate(
        [xl.astype(jnp.bfloat16), jnp.ones((n, 1), jnp.bfloat16)], axis=1)

    nj = n // _TJ
    ni = n // _TI

    for it in range(ni):
        ii = it * _TI
        oacc = None
        for jt in range(nj):
            jj = jt * _TJ
            accb = jnp.zeros((_TJ, _TI), jnp.bfloat16)
            for d in range(dh):
                t = xlab[jj:jj + _TJ, d:d + 1] + xrab[d:d + 1, ii:ii + _TI]
                accb = accb + jnp.abs(t) * jnp.sign(att[0, d]).astype(
                    jnp.bfloat16)
            s = (al2[jj:jj + _TJ] + mrow[:, ii:ii + _TI]
                 + accb.astype(jnp.float32))
            eb = (jnp.exp(s).astype(jnp.bfloat16)
                  * adj_ref[0, jj:jj + _TJ, ii:ii + _TI])
            op = lax.dot_general(eb, xl_aug[jj:jj + _TJ],
                                 (((0,), (0,)), ((), ())),
                                 preferred_element_type=jnp.float32)
            oacc = op if oacc is None else oacc + op
        rec = 1.0 / (oacc[:, dh:dh + 1] + 1e-30)
        out_ref[0, 0, ii:ii + _TI, :] = oacc[:, :dh] * rec + bias_ref[0]


def kernel(x, adj, Wl, Wr, att, bias):
    b, n, in_dim = x.shape
    heads, dh = att.shape

    madj = (adj != 0).astype(jnp.bfloat16)
    wl = Wl.reshape(in_dim, heads, dh).transpose(1, 0, 2)
    wr = Wr.reshape(in_dim, heads, dh).transpose(1, 0, 2)
    wlr = jnp.concatenate([wl, wr], axis=2)
    attr = att.reshape(heads, 1, dh)
    biasr = bias.reshape(heads, 1, dh)

    out = pl.pallas_call(
        _gat_body,
        grid=(b, heads),
        in_specs=[
            pl.BlockSpec((1, n, in_dim), lambda bb, h: (bb, 0, 0)),
            pl.BlockSpec((1, n, n), lambda bb, h: (bb, 0, 0)),
            pl.BlockSpec((1, in_dim, 2 * dh), lambda bb, h: (h, 0, 0)),
            pl.BlockSpec((1, 1, dh), lambda bb, h: (h, 0, 0)),
            pl.BlockSpec((1, 1, dh), lambda bb, h: (h, 0, 0)),
        ],
        out_specs=pl.BlockSpec((1, 1, n, dh), lambda bb, h: (bb, h, 0, 0)),
        out_shape=jax.ShapeDtypeStruct((b, heads, n, dh), jnp.float32),
        compiler_params=pltpu.CompilerParams(
            dimension_semantics=("parallel", "parallel")),
    )(x, madj, wlr, attr, biasr)

    return out.transpose(0, 2, 1, 3).reshape(b, n, heads * dh)


# speedup vs baseline: 1.0945x; 1.0945x over previous
import jax
import jax.numpy as jnp
from jax import lax
from jax.experimental import pallas as pl
from jax.experimental.pallas import tpu as pltpu

_NEG = -1e30
_TJ = 128
_TI = 256


def _gat_body(x_ref, xt_ref, adj_ref, wl_ref, wlt_ref, wrt_ref, att_ref,
              attc_ref, bias_ref, out_ref, s_scr):
    n = x_ref.shape[1]
    dh = wl_ref.shape[2]
    x = x_ref[0]
    xt = xt_ref[0]
    wl = wl_ref[0]
    wlt = wlt_ref[0]
    wrt = wrt_ref[0]
    att = att_ref[0]
    attc = attc_ref[0]

    xl = jnp.dot(x, wl, preferred_element_type=jnp.float32)
    xlt = jnp.dot(wlt, xt, preferred_element_type=jnp.float32)
    xrat = jnp.dot(wrt, xt, preferred_element_type=jnp.float32)

    xlaf = xl * (0.4 * att)
    xrabf = xrat * (0.4 * attc)
    xla = xlaf.astype(jnp.bfloat16)
    xrab = xrabf.astype(jnp.bfloat16)
    al2 = 1.5 * jnp.sum(xla, axis=1, keepdims=True)
    ar2 = 1.5 * jnp.sum(xrab, axis=0, keepdims=True)

    nj = n // _TJ
    ni = n // _TI

    pmax = []
    for it in range(ni):
        ii = it * _TI
        pm = None
        for jt in range(nj):
            jj = jt * _TJ
            accb = jnp.zeros((_TJ, _TI), jnp.bfloat16)
            for d in range(dh):
                t = xla[jj:jj + _TJ, d:d + 1] + xrab[d:d + 1, ii:ii + _TI]
                accb = accb + jnp.abs(t) * jnp.sign(att[0, d]).astype(jnp.bfloat16)
            acc = (al2[jj:jj + _TJ] + ar2[:, ii:ii + _TI]
                   + accb.astype(jnp.float32))
            m = adj_ref[0, jj:jj + _TJ, ii:ii + _TI] != 0
            acc = jnp.where(m, acc, _NEG)
            s_scr[jj:jj + _TJ, ii:ii + _TI] = acc
            t_pm = jnp.max(acc, axis=0, keepdims=True)
            pm = t_pm if pm is None else jnp.maximum(pm, t_pm)
        pmax.append(pm)

    recips = []
    for it in range(ni):
        ii = it * _TI
        amax = jnp.where(pmax[it] > 0.5 * _NEG, pmax[it], 0.0)
        den = None
        for jt in range(nj):
            jj = jt * _TJ
            e = jnp.exp(s_scr[jj:jj + _TJ, ii:ii + _TI] - amax)
            s_scr[jj:jj + _TJ, ii:ii + _TI] = e
            t_den = jnp.sum(e, axis=0, keepdims=True)
            den = t_den if den is None else den + t_den
        recips.append(1.0 / (den + 1e-16))
    recip = jnp.concatenate(recips, axis=1)

    ex = s_scr[...]
    out_t = jnp.dot(xlt, ex, preferred_element_type=jnp.float32)
    out_ref[0, 0] = out_t * recip + bias_ref[0]


def kernel(x, adj, Wl, Wr, att, bias):
    b, n, in_dim = x.shape
    heads, dh = att.shape

    xt = x.transpose(0, 2, 1)
    adj8 = (adj != 0).astype(jnp.int8)
    wl = Wl.reshape(in_dim, heads, dh).transpose(1, 0, 2)
    wlt = Wl.reshape(in_dim, heads, dh).transpose(1, 2, 0)
    wrt = Wr.reshape(in_dim, heads, dh).transpose(1, 2, 0)
    attr = att.reshape(heads, 1, dh)
    attc = att.reshape(heads, dh, 1)
    biasc = bias.reshape(heads, dh, 1)

    out = pl.pallas_call(
        _gat_body,
        grid=(b, heads),
        in_specs=[
            pl.BlockSpec((1, n, in_dim), lambda bb, h: (bb, 0, 0)),
            pl.BlockSpec((1, in_dim, n), lambda bb, h: (bb, 0, 0)),
            pl.BlockSpec((1, n, n), lambda bb, h: (bb, 0, 0)),
            pl.BlockSpec((1, in_dim, dh), lambda bb, h: (h, 0, 0)),
            pl.BlockSpec((1, dh, in_dim), lambda bb, h: (h, 0, 0)),
            pl.BlockSpec((1, dh, in_dim), lambda bb, h: (h, 0, 0)),
            pl.BlockSpec((1, 1, dh), lambda bb, h: (h, 0, 0)),
            pl.BlockSpec((1, dh, 1), lambda bb, h: (h, 0, 0)),
            pl.BlockSpec((1, dh, 1), lambda bb, h: (h, 0, 0)),
        ],
        out_specs=pl.BlockSpec((1, 1, dh, n), lambda bb, h: (bb, h, 0, 0)),
        out_shape=jax.ShapeDtypeStruct((b, heads, dh, n), jnp.float32),
        scratch_shapes=[pltpu.VMEM((n, n), jnp.float32)],
        compiler_params=pltpu.CompilerParams(
            dimension_semantics=("parallel", "parallel")),
    )(x, xt, adj8, wl, wlt, wrt, attr, attc, biasc)

    return out.transpose(0, 3, 1, 2).reshape(b, n, heads * dh)


# speedup vs baseline: 1.1056x; 1.0101x over previous
import jax
import jax.numpy as jnp
from jax import lax
from jax.experimental import pallas as pl
from jax.experimental.pallas import tpu as pltpu

_NEG = -1e30
_TJ = 128
_TI = 256


def _gat_body(x_ref, xt_ref, adj_ref, wl_ref, wlt_ref, wrt_ref, att_ref,
              attc_ref, bias_ref, out_ref, s_scr, e_scr):
    n = x_ref.shape[1]
    dh = wl_ref.shape[2]
    x = x_ref[0]
    xt = xt_ref[0]
    wl = wl_ref[0]
    wlt = wlt_ref[0]
    wrt = wrt_ref[0]
    att = att_ref[0]
    attc = attc_ref[0]

    xl = jnp.dot(x, wl, preferred_element_type=jnp.float32)
    xlt = jnp.dot(wlt, xt, preferred_element_type=jnp.float32)
    xrat = jnp.dot(wrt, xt, preferred_element_type=jnp.float32)

    xlaf = xl * (0.4 * att)
    xrabf = xrat * (0.4 * attc)
    xla = xlaf.astype(jnp.bfloat16)
    xrab = xrabf.astype(jnp.bfloat16)
    al2 = 1.5 * jnp.sum(xla, axis=1, keepdims=True)
    ar2 = 1.5 * jnp.sum(xrab, axis=0, keepdims=True)

    nj = n // _TJ
    ni = n // _TI

    pmax = []
    for it in range(ni):
        ii = it * _TI
        pm = None
        for jt in range(nj):
            jj = jt * _TJ
            accb = jnp.zeros((_TJ, _TI), jnp.bfloat16)
            for d in range(dh):
                t = xla[jj:jj + _TJ, d:d + 1] + xrab[d:d + 1, ii:ii + _TI]
                accb = accb + jnp.abs(t) * jnp.sign(att[0, d]).astype(jnp.bfloat16)
            acc = (al2[jj:jj + _TJ] + ar2[:, ii:ii + _TI]
                   + accb.astype(jnp.float32))
            s_scr[jj:jj + _TJ, ii:ii + _TI] = acc
            t_pm = jnp.max(acc, axis=0, keepdims=True)
            pm = t_pm if pm is None else jnp.maximum(pm, t_pm)
        pmax.append(pm)

    recips = []
    for it in range(ni):
        ii = it * _TI
        amax = pmax[it]
        den = None
        for jt in range(nj):
            jj = jt * _TJ
            e = (jnp.exp(s_scr[jj:jj + _TJ, ii:ii + _TI] - amax)
                 .astype(jnp.bfloat16) * adj_ref[0, jj:jj + _TJ, ii:ii + _TI])
            e_scr[jj:jj + _TJ, ii:ii + _TI] = e
            t_den = jnp.sum(e.astype(jnp.float32), axis=0, keepdims=True)
            den = t_den if den is None else den + t_den
        recips.append(1.0 / (den + 1e-30))
    recip = jnp.concatenate(recips, axis=1)

    ex = e_scr[...]
    out_t = jnp.dot(xlt.astype(jnp.bfloat16), ex,
                    preferred_element_type=jnp.float32)
    out_ref[0, 0] = out_t * recip + bias_ref[0]


def kernel(x, adj, Wl, Wr, att, bias):
    b, n, in_dim = x.shape
    heads, dh = att.shape

    xt = x.transpose(0, 2, 1)
    adj8 = (adj != 0).astype(jnp.bfloat16)
    wl = Wl.reshape(in_dim, heads, dh).transpose(1, 0, 2)
    wlt = Wl.reshape(in_dim, heads, dh).transpose(1, 2, 0)
    wrt = Wr.reshape(in_dim, heads, dh).transpose(1, 2, 0)
    attr = att.reshape(heads, 1, dh)
    attc = att.reshape(heads, dh, 1)
    biasc = bias.reshape(heads, dh, 1)

    out = pl.pallas_call(
        _gat_body,
        grid=(b, heads),
        in_specs=[
            pl.BlockSpec((1, n, in_dim), lambda bb, h: (bb, 0, 0)),
            pl.BlockSpec((1, in_dim, n), lambda bb, h: (bb, 0, 0)),
            pl.BlockSpec((1, n, n), lambda bb, h: (bb, 0, 0)),
            pl.BlockSpec((1, in_dim, dh), lambda bb, h: (h, 0, 0)),
            pl.BlockSpec((1, dh, in_dim), lambda bb, h: (h, 0, 0)),
            pl.BlockSpec((1, dh, in_dim), lambda bb, h: (h, 0, 0)),
            pl.BlockSpec((1, 1, dh), lambda bb, h: (h, 0, 0)),
            pl.BlockSpec((1, dh, 1), lambda bb, h: (h, 0, 0)),
            pl.BlockSpec((1, dh, 1), lambda bb, h: (h, 0, 0)),
        ],
        out_specs=pl.BlockSpec((1, 1, dh, n), lambda bb, h: (bb, h, 0, 0)),
        out_shape=jax.ShapeDtypeStruct((b, heads, dh, n), jnp.float32),
        scratch_shapes=[pltpu.VMEM((n, n), jnp.float32),
                        pltpu.VMEM((n, n), jnp.bfloat16)],
        compiler_params=pltpu.CompilerParams(
            dimension_semantics=("parallel", "parallel")),
    )(x, xt, adj8, wl, wlt, wrt, attr, attc, biasc)

    return out.transpose(0, 3, 1, 2).reshape(b, n, heads * dh)


# speedup vs baseline: 1.1349x; 1.0264x over previous
import jax
import jax.numpy as jnp
from jax import lax
from jax.experimental import pallas as pl
from jax.experimental.pallas import tpu as pltpu

_NEG = -1e30
_TJ = 128
_TI = 256


def _gat_body(x_ref, xt_ref, adj_ref, wl_ref, wlt_ref, wrt_ref, att_ref,
              attc_ref, bias_ref, out_ref, s_scr, e_scr):
    n = x_ref.shape[1]
    dh = wl_ref.shape[2]
    x = x_ref[0]
    xt = xt_ref[0]
    wl = wl_ref[0]
    wlt = wlt_ref[0]
    wrt = wrt_ref[0]
    att = att_ref[0]
    attc = attc_ref[0]

    xl = jnp.dot(x, wl, preferred_element_type=jnp.float32)
    xlt = jnp.dot(wlt, xt, preferred_element_type=jnp.float32)
    xrat = jnp.dot(wrt, xt, preferred_element_type=jnp.float32)

    _l2e = 1.4426950408889634
    xlaf = xl * ((0.4 * _l2e) * att)
    xrabf = xrat * ((0.4 * _l2e) * attc)
    xla = xlaf.astype(jnp.bfloat16)
    xrab = xrabf.astype(jnp.bfloat16)
    al2 = (1.5 * jnp.sum(xla, axis=1, keepdims=True)).astype(jnp.bfloat16)
    ar2 = (1.5 * jnp.sum(xrab, axis=0, keepdims=True)).astype(jnp.bfloat16)

    nj = n // _TJ
    ni = n // _TI

    pmax = []
    for it in range(ni):
        ii = it * _TI
        pm = None
        for jt in range(nj):
            jj = jt * _TJ
            accb = jnp.zeros((_TJ, _TI), jnp.bfloat16)
            for d in range(dh):
                t = xla[jj:jj + _TJ, d:d + 1] + xrab[d:d + 1, ii:ii + _TI]
                accb = accb + jnp.abs(t) * jnp.sign(att[0, d]).astype(jnp.bfloat16)
            acc = (al2[jj:jj + _TJ] + ar2[:, ii:ii + _TI]) + accb
            s_scr[jj:jj + _TJ, ii:ii + _TI] = acc
            t_pm = jnp.max(acc, axis=0, keepdims=True)
            pm = t_pm if pm is None else jnp.maximum(pm, t_pm)
        pmax.append(pm)

    recips = []
    for it in range(ni):
        ii = it * _TI
        amax = pmax[it]
        den = None
        for jt in range(nj):
            jj = jt * _TJ
            sb = s_scr[jj:jj + _TJ, ii:ii + _TI] - amax
            e = (jnp.exp2(sb.astype(jnp.float32)).astype(jnp.bfloat16)
                 * adj_ref[0, jj:jj + _TJ, ii:ii + _TI])
            e_scr[jj:jj + _TJ, ii:ii + _TI] = e
            t_den = jnp.sum(e.astype(jnp.float32), axis=0, keepdims=True)
            den = t_den if den is None else den + t_den
        recips.append(1.0 / (den + 1e-30))
    recip = jnp.concatenate(recips, axis=1)

    ex = e_scr[...]
    out_t = jnp.dot(xlt.astype(jnp.bfloat16), ex,
                    preferred_element_type=jnp.float32)
    out_ref[0, 0] = out_t * recip + bias_ref[0]


def kernel(x, adj, Wl, Wr, att, bias):
    b, n, in_dim = x.shape
    heads, dh = att.shape

    xt = x.transpose(0, 2, 1)
    adj8 = (adj != 0).astype(jnp.bfloat16)
    wl = Wl.reshape(in_dim, heads, dh).transpose(1, 0, 2)
    wlt = Wl.reshape(in_dim, heads, dh).transpose(1, 2, 0)
    wrt = Wr.reshape(in_dim, heads, dh).transpose(1, 2, 0)
    attr = att.reshape(heads, 1, dh)
    attc = att.reshape(heads, dh, 1)
    biasc = bias.reshape(heads, dh, 1)

    out = pl.pallas_call(
        _gat_body,
        grid=(b, heads),
        in_specs=[
            pl.BlockSpec((1, n, in_dim), lambda bb, h: (bb, 0, 0)),
            pl.BlockSpec((1, in_dim, n), lambda bb, h: (bb, 0, 0)),
            pl.BlockSpec((1, n, n), lambda bb, h: (bb, 0, 0)),
            pl.BlockSpec((1, in_dim, dh), lambda bb, h: (h, 0, 0)),
            pl.BlockSpec((1, dh, in_dim), lambda bb, h: (h, 0, 0)),
            pl.BlockSpec((1, dh, in_dim), lambda bb, h: (h, 0, 0)),
            pl.BlockSpec((1, 1, dh), lambda bb, h: (h, 0, 0)),
            pl.BlockSpec((1, dh, 1), lambda bb, h: (h, 0, 0)),
            pl.BlockSpec((1, dh, 1), lambda bb, h: (h, 0, 0)),
        ],
        out_specs=pl.BlockSpec((1, 1, dh, n), lambda bb, h: (bb, h, 0, 0)),
        out_shape=jax.ShapeDtypeStruct((b, heads, dh, n), jnp.float32),
        scratch_shapes=[pltpu.VMEM((n, n), jnp.bfloat16),
                        pltpu.VMEM((n, n), jnp.bfloat16)],
        compiler_params=pltpu.CompilerParams(
            dimension_semantics=("parallel", "parallel")),
    )(x, xt, adj8, wl, wlt, wrt, attr, attc, biasc)

    return out.transpose(0, 3, 1, 2).reshape(b, n, heads * dh)


# speedup vs baseline: 1.1818x; 1.0414x over previous
import jax
import jax.numpy as jnp
from jax import lax
from jax.experimental import pallas as pl
from jax.experimental.pallas import tpu as pltpu

_NEG = -1e30
_TJ = 128
_TI = 256


def _gat_body(x_ref, xt_ref, adj_ref, wl_ref, wlt_ref, wrt_ref, att_ref,
              attc_ref, bias_ref, out_ref, s_scr, e_scr):
    n = x_ref.shape[1]
    dh = wl_ref.shape[2]
    x = x_ref[0]
    xt = xt_ref[0]
    wl = wl_ref[0]
    wlt = wlt_ref[0]
    wrt = wrt_ref[0]
    att = att_ref[0]
    attc = attc_ref[0]

    xl = jnp.dot(x, wl, preferred_element_type=jnp.float32)
    xlt = jnp.dot(wlt, xt, preferred_element_type=jnp.float32)
    xrat = jnp.dot(wrt, xt, preferred_element_type=jnp.float32)

    _l2e = 1.4426950408889634
    xlaf = xl * ((0.4 * _l2e) * att)
    xrabf = xrat * ((0.4 * _l2e) * attc)
    xlaM = (xlaf * -2.0).astype(jnp.bfloat16)
    xrabM = (xrabf * 2.0).astype(jnp.bfloat16)
    sgnr = jnp.sign(att)
    sgnc = jnp.sign(attc)
    colterm = ((-0.75) * jnp.sum(xlaM, axis=1, keepdims=True)
               + (-0.5) * jnp.sum(xlaM * sgnr, axis=1, keepdims=True)
               ).astype(jnp.bfloat16)
    rowterm = (0.75 * jnp.sum(xrabM, axis=0, keepdims=True)
               - 0.5 * jnp.sum(xrabM * sgnc, axis=0, keepdims=True)
               ).astype(jnp.bfloat16)

    nj = n // _TJ
    ni = n // _TI

    pmax = []
    for it in range(ni):
        ii = it * _TI
        pm = None
        for jt in range(nj):
            jj = jt * _TJ
            accb = jnp.zeros((_TJ, _TI), jnp.bfloat16)
            for d in range(dh):
                u = jnp.maximum(xlaM[jj:jj + _TJ, d:d + 1],
                                xrabM[d:d + 1, ii:ii + _TI])
                accb = accb + u * jnp.sign(att[0, d]).astype(jnp.bfloat16)
            acc = (colterm[jj:jj + _TJ] + rowterm[:, ii:ii + _TI]) + accb
            s_scr[jj:jj + _TJ, ii:ii + _TI] = acc
            t_pm = jnp.max(acc, axis=0, keepdims=True)
            pm = t_pm if pm is None else jnp.maximum(pm, t_pm)
        pmax.append(pm)

    recips = []
    for it in range(ni):
        ii = it * _TI
        amax = pmax[it]
        den = None
        for jt in range(nj):
            jj = jt * _TJ
            sb = s_scr[jj:jj + _TJ, ii:ii + _TI] - amax
            e = (jnp.exp2(sb.astype(jnp.float32)).astype(jnp.bfloat16)
                 * adj_ref[0, jj:jj + _TJ, ii:ii + _TI])
            e_scr[jj:jj + _TJ, ii:ii + _TI] = e
            t_den = jnp.sum(e.astype(jnp.float32), axis=0, keepdims=True)
            den = t_den if den is None else den + t_den
        recips.append(1.0 / (den + 1e-30))
    recip = jnp.concatenate(recips, axis=1)

    ex = e_scr[...]
    out_t = jnp.dot(xlt.astype(jnp.bfloat16), ex,
                    preferred_element_type=jnp.float32)
    out_ref[0, 0] = out_t * recip + bias_ref[0]


def kernel(x, adj, Wl, Wr, att, bias):
    b, n, in_dim = x.shape
    heads, dh = att.shape

    xt = x.transpose(0, 2, 1)
    adj8 = (adj != 0).astype(jnp.bfloat16)
    wl = Wl.reshape(in_dim, heads, dh).transpose(1, 0, 2)
    wlt = Wl.reshape(in_dim, heads, dh).transpose(1, 2, 0)
    wrt = Wr.reshape(in_dim, heads, dh).transpose(1, 2, 0)
    attr = att.reshape(heads, 1, dh)
    attc = att.reshape(heads, dh, 1)
    biasc = bias.reshape(heads, dh, 1)

    out = pl.pallas_call(
        _gat_body,
        grid=(b, heads),
        in_specs=[
            pl.BlockSpec((1, n, in_dim), lambda bb, h: (bb, 0, 0)),
            pl.BlockSpec((1, in_dim, n), lambda bb, h: (bb, 0, 0)),
            pl.BlockSpec((1, n, n), lambda bb, h: (bb, 0, 0)),
            pl.BlockSpec((1, in_dim, dh), lambda bb, h: (h, 0, 0)),
            pl.BlockSpec((1, dh, in_dim), lambda bb, h: (h, 0, 0)),
            pl.BlockSpec((1, dh, in_dim), lambda bb, h: (h, 0, 0)),
            pl.BlockSpec((1, 1, dh), lambda bb, h: (h, 0, 0)),
            pl.BlockSpec((1, dh, 1), lambda bb, h: (h, 0, 0)),
            pl.BlockSpec((1, dh, 1), lambda bb, h: (h, 0, 0)),
        ],
        out_specs=pl.BlockSpec((1, 1, dh, n), lambda bb, h: (bb, h, 0, 0)),
        out_shape=jax.ShapeDtypeStruct((b, heads, dh, n), jnp.float32),
        scratch_shapes=[pltpu.VMEM((n, n), jnp.bfloat16),
                        pltpu.VMEM((n, n), jnp.bfloat16)],
        compiler_params=pltpu.CompilerParams(
            dimension_semantics=("parallel", "parallel")),
    )(x, xt, adj8, wl, wlt, wrt, attr, attc, biasc)

    return out.transpose(0, 3, 1, 2).reshape(b, n, heads * dh)
